# baseline (speedup 1.0000x reference)
import jax
import jax.numpy as jnp
from jax.experimental import pallas as pl

_TOP_K = 8
_N_EXPERTS = 64


def _gate_kernel(x_ref, w_ref, idx_ref, wgt_ref):
    x = x_ref[...]
    w = w_ref[...]
    logits = jax.lax.dot_general(
        x, w, dimension_numbers=(((1,), (1,)), ((), ())),
        preferred_element_type=jnp.float32)
    lane = jax.lax.broadcasted_iota(jnp.int32, logits.shape, 1)
    vals = logits
    sel_v = []
    sel_i = []
    for _ in range(_TOP_K):
        m = jnp.max(vals, axis=1, keepdims=True)
        cand = jnp.where(vals == m, lane, _N_EXPERTS)
        a = jnp.min(cand, axis=1, keepdims=True)
        sel_v.append(m)
        sel_i.append(a)
        vals = jnp.where(lane == a, -jnp.inf, vals)
    top_v = jnp.concatenate(sel_v, axis=1)
    top_i = jnp.concatenate(sel_i, axis=1)
    e = jnp.exp(top_v - top_v[:, 0:1])
    wgt = e / jnp.sum(e, axis=1, keepdims=True)
    idx_ref[...] = top_i
    wgt_ref[...] = wgt


def kernel(hidden_states, weight):
    bsz, seq_len, dim = hidden_states.shape
    n_tokens = bsz * seq_len
    x = hidden_states.reshape(n_tokens, dim)
    block = 512
    idx, wgt = pl.pallas_call(
        _gate_kernel,
        grid=(n_tokens // block,),
        in_specs=[
            pl.BlockSpec((block, dim), lambda i: (i, 0)),
            pl.BlockSpec((_N_EXPERTS, dim), lambda i: (0, 0)),
        ],
        out_specs=[
            pl.BlockSpec((block, _TOP_K), lambda i: (i, 0)),
            pl.BlockSpec((block, _TOP_K), lambda i: (i, 0)),
        ],
        out_shape=[
            jax.ShapeDtypeStruct((n_tokens, _TOP_K), jnp.int32),
            jax.ShapeDtypeStruct((n_tokens, _TOP_K), jnp.float32),
        ],
    )(x, weight)
    aux_loss = jnp.asarray(0.0, dtype=hidden_states.dtype)
    return idx, wgt.astype(hidden_states.dtype), aux_loss


# speedup vs baseline: 1.6821x; 1.6821x over previous
import jax
import jax.numpy as jnp
from jax.experimental import pallas as pl

_TOP_K = 8
_N_EXPERTS = 64


def _gate_kernel(x_ref, w_ref, idx_ref, wgt_ref):
    x = x_ref[...]
    w = w_ref[...]
    logits = jax.lax.dot_general(
        w, x, dimension_numbers=(((1,), (1,)), ((), ())),
        preferred_element_type=jnp.float32)
    row = jax.lax.broadcasted_iota(jnp.int32, logits.shape, 0)
    vals = logits
    sel_v = []
    sel_i = []
    for _ in range(_TOP_K):
        m = jnp.max(vals, axis=0, keepdims=True)
        cand = jnp.where(vals == m, row, _N_EXPERTS)
        a = jnp.min(cand, axis=0, keepdims=True)
        sel_v.append(m)
        sel_i.append(a)
        vals = jnp.where(row == a, -jnp.inf, vals)
    top_v = jnp.concatenate(sel_v, axis=0)
    top_i = jnp.concatenate(sel_i, axis=0)
    e = jnp.exp(top_v - top_v[0:1, :])
    wgt = e / jnp.sum(e, axis=0, keepdims=True)
    idx_ref[...] = top_i
    wgt_ref[...] = wgt


def kernel(hidden_states, weight):
    bsz, seq_len, dim = hidden_states.shape
    n_tokens = bsz * seq_len
    x = hidden_states.reshape(n_tokens, dim)
    block = 512
    idx_t, wgt_t = pl.pallas_call(
        _gate_kernel,
        grid=(n_tokens // block,),
        in_specs=[
            pl.BlockSpec((block, dim), lambda i: (i, 0)),
            pl.BlockSpec((_N_EXPERTS, dim), lambda i: (0, 0)),
        ],
        out_specs=[
            pl.BlockSpec((_TOP_K, block), lambda i: (0, i)),
            pl.BlockSpec((_TOP_K, block), lambda i: (0, i)),
        ],
        out_shape=[
            jax.ShapeDtypeStruct((_TOP_K, n_tokens), jnp.int32),
            jax.ShapeDtypeStruct((_TOP_K, n_tokens), jnp.float32),
        ],
    )(x, weight)
    aux_loss = jnp.asarray(0.0, dtype=hidden_states.dtype)
    return idx_t.T, wgt_t.T.astype(hidden_states.dtype), aux_loss
